# baseline (speedup 1.0000x reference)
import functools

import jax
import jax.numpy as jnp
from jax import lax
from jax.experimental import pallas as pl
from jax.experimental.pallas import tpu as pltpu
from jax.experimental.pallas import tpu_sc as plsc

CONTEXT_LENGTH = 8192
EMBEDDING_DIM = 1024
BATCH = 4
ROWS = BATCH * CONTEXT_LENGTH

SEQ_BLOCK = 2048
SC_START = 6144
SC_SEQ = CONTEXT_LENGTH - SC_START

NUM_CORES = 2
NUM_SUBCORES = 16
NUM_WORKERS = NUM_CORES * NUM_SUBCORES
SEQ_PER_WORKER = SC_SEQ // NUM_WORKERS
CHUNK_ROWS = 16
NUM_CHUNKS = SEQ_PER_WORKER // CHUNK_ROWS

_mesh = plsc.VectorSubcoreMesh(core_axis_name="c", subcore_axis_name="s")


@functools.partial(
    pl.kernel,
    mesh=_mesh,
    out_type=jax.ShapeDtypeStruct((ROWS, EMBEDDING_DIM), jnp.float32),
    scratch_types=(
        [pltpu.VMEM((CHUNK_ROWS, EMBEDDING_DIM), jnp.float32) for _ in range(BATCH)]
        + [pltpu.VMEM((CHUNK_ROWS, EMBEDDING_DIM), jnp.float32)]
        + [pltpu.SemaphoreType.DMA for _ in range(2 * BATCH + 1)]
    ),
)
def _sc_add(x_hbm, pos_hbm, out_hbm, xv0, xv1, xv2, xv3, pv,
            l0, l1, l2, l3, s0, s1, s2, s3, psem):
    bufs = (xv0, xv1, xv2, xv3)
    lsems = (l0, l1, l2, l3)
    ssems = (s0, s1, s2, s3)
    wid = lax.axis_index("s") * NUM_CORES + lax.axis_index("c")
    seq_base = SC_START + wid * SEQ_PER_WORKER

    @pl.loop(0, NUM_CHUNKS)
    def _chunk(ci):
        prow = seq_base + ci * CHUNK_ROWS
        pcopy = pltpu.async_copy(pos_hbm.at[pl.ds(prow, CHUNK_ROWS)], pv, psem)
        for b in range(BATCH):
            xrow = b * CONTEXT_LENGTH + prow

            @pl.when(ci > 0)
            def _drain():
                pltpu.make_async_copy(
                    bufs[b], out_hbm.at[pl.ds(xrow - CHUNK_ROWS, CHUNK_ROWS)], ssems[b]
                ).wait()

            pltpu.async_copy(x_hbm.at[pl.ds(xrow, CHUNK_ROWS)], bufs[b], lsems[b])
        pcopy.wait()
        for b in range(BATCH):
            xrow = b * CONTEXT_LENGTH + prow
            buf = bufs[b]
            pltpu.make_async_copy(
                x_hbm.at[pl.ds(xrow, CHUNK_ROWS)], buf, lsems[b]
            ).wait()

            @pl.loop(0, CHUNK_ROWS)
            def _row(r):
                @plsc.parallel_loop(0, EMBEDDING_DIM, step=16, unroll=16)
                def _add(i):
                    s = pl.ds(i, 16)
                    buf[r, s] = buf[r, s] + pv[r, s]

            pltpu.async_copy(buf, out_hbm.at[pl.ds(xrow, CHUNK_ROWS)], ssems[b])

    last = seq_base + (NUM_CHUNKS - 1) * CHUNK_ROWS
    for b in range(BATCH):
        pltpu.make_async_copy(
            bufs[b], out_hbm.at[pl.ds(b * CONTEXT_LENGTH + last, CHUNK_ROWS)], ssems[b]
        ).wait()


def _tc_add_kernel(x_ref, pos_ref, prev_ref, out_ref):
    del prev_ref
    out_ref[...] = x_ref[...] + pos_ref[...][None]


def _tc_fill(x, pos_table, sc_out3):
    grid = (SC_START // SEQ_BLOCK, BATCH)
    return pl.pallas_call(
        _tc_add_kernel,
        grid=grid,
        in_specs=[
            pl.BlockSpec((1, SEQ_BLOCK, EMBEDDING_DIM), lambda i, b: (b, i, 0)),
            pl.BlockSpec((SEQ_BLOCK, EMBEDDING_DIM), lambda i, b: (i, 0)),
            pl.BlockSpec((1, 8, 128), lambda i, b: (0, 0, 0)),
        ],
        out_specs=pl.BlockSpec((1, SEQ_BLOCK, EMBEDDING_DIM), lambda i, b: (b, i, 0)),
        out_shape=jax.ShapeDtypeStruct(x.shape, x.dtype),
        input_output_aliases={2: 0},
    )(x, pos_table, sc_out3)


def kernel(x, pos_table):
    x2 = x.reshape(ROWS, EMBEDDING_DIM)
    sc_out = _sc_add(x2, pos_table)
    out = _tc_fill(x, pos_table, sc_out.reshape(x.shape))
    return out


# speedup vs baseline: 1.0339x; 1.0339x over previous
import functools

import jax
import jax.numpy as jnp
from jax import lax
from jax.experimental import pallas as pl
from jax.experimental.pallas import tpu as pltpu
from jax.experimental.pallas import tpu_sc as plsc

CONTEXT_LENGTH = 8192
EMBEDDING_DIM = 1024
BATCH = 4
ROWS = BATCH * CONTEXT_LENGTH

SEQ_BLOCK = 1024
SC_START = 7168
SC_SEQ = CONTEXT_LENGTH - SC_START

NUM_CORES = 2
NUM_SUBCORES = 16
NUM_WORKERS = NUM_CORES * NUM_SUBCORES
SEQ_PER_WORKER = SC_SEQ // NUM_WORKERS
CHUNK_ROWS = 16
NUM_CHUNKS = SEQ_PER_WORKER // CHUNK_ROWS

_mesh = plsc.VectorSubcoreMesh(core_axis_name="c", subcore_axis_name="s")


@functools.partial(
    pl.kernel,
    mesh=_mesh,
    out_type=jax.ShapeDtypeStruct((ROWS, EMBEDDING_DIM), jnp.float32),
    scratch_types=(
        [pltpu.VMEM((CHUNK_ROWS, EMBEDDING_DIM), jnp.float32) for _ in range(BATCH)]
        + [pltpu.VMEM((CHUNK_ROWS, EMBEDDING_DIM), jnp.float32)]
        + [pltpu.SemaphoreType.DMA for _ in range(2 * BATCH + 1)]
    ),
)
def _sc_add(x_hbm, pos_hbm, out_hbm, xv0, xv1, xv2, xv3, pv,
            l0, l1, l2, l3, s0, s1, s2, s3, psem):
    bufs = (xv0, xv1, xv2, xv3)
    lsems = (l0, l1, l2, l3)
    ssems = (s0, s1, s2, s3)
    wid = lax.axis_index("s") * NUM_CORES + lax.axis_index("c")
    seq_base = SC_START + wid * SEQ_PER_WORKER

    @pl.loop(0, NUM_CHUNKS)
    def _chunk(ci):
        prow = seq_base + ci * CHUNK_ROWS
        pcopy = pltpu.async_copy(pos_hbm.at[pl.ds(prow, CHUNK_ROWS)], pv, psem)
        for b in range(BATCH):
            xrow = b * CONTEXT_LENGTH + prow

            @pl.when(ci > 0)
            def _drain():
                pltpu.make_async_copy(
                    bufs[b], out_hbm.at[pl.ds(xrow - CHUNK_ROWS, CHUNK_ROWS)], ssems[b]
                ).wait()

            pltpu.async_copy(x_hbm.at[pl.ds(xrow, CHUNK_ROWS)], bufs[b], lsems[b])
        pcopy.wait()
        for b in range(BATCH):
            xrow = b * CONTEXT_LENGTH + prow
            buf = bufs[b]
            pltpu.make_async_copy(
                x_hbm.at[pl.ds(xrow, CHUNK_ROWS)], buf, lsems[b]
            ).wait()

            @pl.loop(0, CHUNK_ROWS)
            def _row(r):
                @plsc.parallel_loop(0, EMBEDDING_DIM, step=16, unroll=16)
                def _add(i):
                    s = pl.ds(i, 16)
                    buf[r, s] = buf[r, s] + pv[r, s]

            pltpu.async_copy(buf, out_hbm.at[pl.ds(xrow, CHUNK_ROWS)], ssems[b])

    last = seq_base + (NUM_CHUNKS - 1) * CHUNK_ROWS
    for b in range(BATCH):
        pltpu.make_async_copy(
            bufs[b], out_hbm.at[pl.ds(b * CONTEXT_LENGTH + last, CHUNK_ROWS)], ssems[b]
        ).wait()


def _tc_add_kernel(x_ref, pos_ref, prev_ref, out_ref):
    del prev_ref
    out_ref[...] = x_ref[...] + pos_ref[...][None]


def _tc_fill(x, pos_table, sc_out3):
    grid = (SC_START // SEQ_BLOCK, BATCH)
    return pl.pallas_call(
        _tc_add_kernel,
        grid=grid,
        in_specs=[
            pl.BlockSpec((1, SEQ_BLOCK, EMBEDDING_DIM), lambda i, b: (b, i, 0)),
            pl.BlockSpec((SEQ_BLOCK, EMBEDDING_DIM), lambda i, b: (i, 0)),
            pl.BlockSpec((1, 8, 128), lambda i, b: (0, 0, 0)),
        ],
        out_specs=pl.BlockSpec((1, SEQ_BLOCK, EMBEDDING_DIM), lambda i, b: (b, i, 0)),
        out_shape=jax.ShapeDtypeStruct(x.shape, x.dtype),
        input_output_aliases={2: 0},
    )(x, pos_table, sc_out3)


def kernel(x, pos_table):
    x2 = x.reshape(ROWS, EMBEDDING_DIM)
    sc_out = _sc_add(x2, pos_table)
    out = _tc_fill(x, pos_table, sc_out.reshape(x.shape))
    return out
